# baseline (speedup 1.0000x reference)
import functools

import jax
import jax.numpy as jnp
from jax import lax
from jax.experimental import pallas as pl
from jax.experimental.pallas import tpu as pltpu
from jax.experimental.pallas import tpu_sc as plsc

B = 16384
D = 64
NC = 2
NS = 16
NW = NC * NS
BW = B // NW
CH = 128
NCH = BW // CH
L = 16


def _sc_body(uix_hbm, pix_hbm, nix_hbm, utab_hbm, itab_hbm, out_hbm,
             uix, pix, nix, ubuf, pbuf, nbuf, obuf, sem):
    c = lax.axis_index("c")
    s = lax.axis_index("s")
    wid = s * NC + c
    base_w = wid * BW
    lane = lax.iota(jnp.int32, L)

    def chunk(ci, carry):
        base = base_w + ci * CH
        pltpu.sync_copy(uix_hbm.at[pl.ds(base, CH)], uix)
        pltpu.sync_copy(pix_hbm.at[pl.ds(base, CH)], pix)
        pltpu.sync_copy(nix_hbm.at[pl.ds(base, CH)], nix)
        cp_u = pltpu.async_copy(utab_hbm.at[uix], ubuf, sem)
        cp_p = pltpu.async_copy(itab_hbm.at[pix], pbuf, sem)
        cp_n = pltpu.async_copy(itab_hbm.at[nix], nbuf, sem)
        cp_u.wait()
        cp_p.wait()
        cp_n.wait()

        def group(g, carry2):
            r0 = g * L
            y = jnp.zeros((L,), jnp.float32)
            for j in range(L):
                r = r0 + j
                acc = None
                for k in range(D // L):
                    u = ubuf[r, pl.ds(k * L, L)]
                    p = pbuf[r, pl.ds(k * L, L)]
                    n = nbuf[r, pl.ds(k * L, L)]
                    t = u * (p - n)
                    acc = t if acc is None else acc + t
                vals = [acc[d] for d in range(L)]
                while len(vals) > 1:
                    vals = [vals[i] + vals[i + 1] for i in range(0, len(vals), 2)]
                y = jnp.where(lane == j, vals[0], y)
            obuf[pl.ds(r0, L)] = 1.0 / (1.0 + jnp.exp(-y))
            return carry2

        lax.fori_loop(0, CH // L, group, 0)
        pltpu.sync_copy(obuf, out_hbm.at[pl.ds(base, CH)])
        return carry

    lax.fori_loop(0, NCH, chunk, 0)


_mesh = plsc.VectorSubcoreMesh(core_axis_name="c", subcore_axis_name="s")

_sc_call = functools.partial(
    pl.kernel,
    out_type=jax.ShapeDtypeStruct((B,), jnp.float32),
    mesh=_mesh,
    compiler_params=pltpu.CompilerParams(use_tc_tiling_on_sc=False),
    scratch_types=[
        pltpu.VMEM((CH,), jnp.int32),
        pltpu.VMEM((CH,), jnp.int32),
        pltpu.VMEM((CH,), jnp.int32),
        pltpu.VMEM((CH, D), jnp.float32),
        pltpu.VMEM((CH, D), jnp.float32),
        pltpu.VMEM((CH, D), jnp.float32),
        pltpu.VMEM((CH,), jnp.float32),
        pltpu.SemaphoreType.DMA,
    ],
)(_sc_body)


def kernel(inputs, user_table, item_table):
    idx = inputs.astype(jnp.int32)
    out = _sc_call(idx[:, 0], idx[:, 1], idx[:, 2], user_table, item_table)
    return out[:, None]


# speedup vs baseline: 2.0505x; 2.0505x over previous
import functools

import jax
import jax.numpy as jnp
from jax import lax
from jax.experimental import pallas as pl
from jax.experimental.pallas import tpu as pltpu
from jax.experimental.pallas import tpu_sc as plsc

B = 16384
D = 64
NC = 2
NS = 16
NW = NC * NS
BW = B // NW
L = 16
RING = 8
G = 8


def _sc_body(uix_hbm, pix_hbm, nix_hbm, utab3, itab3, out_hbm,
             uixv, pixv, nixv, ebuf, obuf, sems):
    c = lax.axis_index("c")
    s = lax.axis_index("s")
    wid = s * NC + c
    base_w = wid * BW
    lane = lax.iota(jnp.int32, L)

    pltpu.sync_copy(uix_hbm.at[pl.ds(base_w, BW)], uixv)
    pltpu.sync_copy(pix_hbm.at[pl.ds(base_w, BW)], pixv)
    pltpu.sync_copy(nix_hbm.at[pl.ds(base_w, BW)], nixv)

    def fire(vb, lj, slot):
        iu = uixv[pl.ds(vb, L)][lj] >> 3
        ip = pixv[pl.ds(vb, L)][lj] >> 3
        inn = nixv[pl.ds(vb, L)][lj] >> 3
        pltpu.async_copy(utab3.at[iu], ebuf.at[slot, 0], sems.at[slot])
        pltpu.async_copy(itab3.at[ip], ebuf.at[slot, 1], sems.at[slot])
        pltpu.async_copy(itab3.at[inn], ebuf.at[slot, 2], sems.at[slot])

    for i in range(RING):
        fire(0, i, i)

    def group(g, carry):
        y = jnp.zeros((L,), jnp.float32)
        iu = uixv[pl.ds(g * L, L)]
        ip = pixv[pl.ds(g * L, L)]
        inn = nixv[pl.ds(g * L, L)]
        for j in range(L):
            slot = j % RING
            for t in range(3):
                pltpu.make_async_copy(
                    utab3.at[0], ebuf.at[slot, t], sems.at[slot]
                ).wait()
            su = iu[j] & (G - 1)
            sp = ip[j] & (G - 1)
            sn = inn[j] & (G - 1)
            acc = None
            for k in range(D // L):
                u = ebuf[slot, 0, su, pl.ds(k * L, L)]
                p = ebuf[slot, 1, sp, pl.ds(k * L, L)]
                n = ebuf[slot, 2, sn, pl.ds(k * L, L)]
                t2 = u * (p - n)
                acc = t2 if acc is None else acc + t2
            vals = [acc[d] for d in range(L)]
            while len(vals) > 1:
                vals = [vals[a] + vals[a + 1]
                        for a in range(0, len(vals), 2)]
            y = jnp.where(lane == j, vals[0], y)

            nxt = g * L + j + RING

            @pl.when(nxt < BW)
            def _():
                vb = (g + (j + RING) // L) * L
                fire(vb, (j + RING) % L, slot)

        obuf[pl.ds(g * L, L)] = 1.0 / (1.0 + jnp.exp(-y))
        return carry

    lax.fori_loop(0, BW // L, group, 0)
    pltpu.sync_copy(obuf, out_hbm.at[pl.ds(base_w, BW)])


_mesh = plsc.VectorSubcoreMesh(core_axis_name="c", subcore_axis_name="s")

_sc_call = functools.partial(
    pl.kernel,
    out_type=jax.ShapeDtypeStruct((B,), jnp.float32),
    mesh=_mesh,
    scratch_types=[
        pltpu.VMEM((BW,), jnp.int32),
        pltpu.VMEM((BW,), jnp.int32),
        pltpu.VMEM((BW,), jnp.int32),
        pltpu.VMEM((RING, 3, G, D), jnp.float32),
        pltpu.VMEM((BW,), jnp.float32),
        pltpu.SemaphoreType.DMA((RING,)),
    ],
)(_sc_body)


def kernel(inputs, user_table, item_table):
    idx = inputs.astype(jnp.int32)
    ut3 = user_table.reshape(125000, G, D)
    it3 = item_table.reshape(125000, G, D)
    out = _sc_call(idx[:, 0], idx[:, 1], idx[:, 2], ut3, it3)
    return out[:, None]
